# Initial kernel scaffold; baseline (speedup 1.0000x reference)
import jax
import jax.numpy as jnp
from jax.experimental import pallas as pl


def kernel(x, edge_index, batch, W1, b1, W2, b2, W3, b3, Wc):
    raise NotImplementedError("write your pallas kernel here")


# speedup vs baseline: 1.0135x; 1.0135x over previous
import functools

import jax
import jax.numpy as jnp
from jax.experimental import pallas as pl

N_NODES = 10000
NP = 10240
N_EDGES = 320000
NUM_GRAPHS = 64
K_POOL = 50
NUM_ITER = 3


def _mm_kernel(w_ref, x_ref, o_ref):
    o_ref[...] = jnp.dot(w_ref[...], x_ref[...],
                         preferred_element_type=jnp.float32)


def _fm_matmul(W, X):
    fo, fi = W.shape
    blk = 1280
    grid = NP // blk
    return pl.pallas_call(
        _mm_kernel,
        grid=(grid,),
        in_specs=[
            pl.BlockSpec((fo, fi), lambda j: (0, 0)),
            pl.BlockSpec((fi, blk), lambda j: (0, j)),
        ],
        out_specs=pl.BlockSpec((fo, blk), lambda j: (0, j)),
        out_shape=jax.ShapeDtypeStruct((fo, NP), jnp.float32),
    )(W, X)


def _squash(s):
    sn = jnp.sum(s * s, axis=-1, keepdims=True)
    return (sn / (1.0 + sn)) * s / jnp.sqrt(sn + 1e-8)


def kernel(x, edge_index, batch, W1, b1, W2, b2, W3, b3, Wc):
    src, dst = edge_index[0], edge_index[1]
    w = (src != dst).astype(jnp.float32)
    deg = jnp.zeros(NP, jnp.float32).at[dst].add(w) + 1.0
    dinv = jax.lax.rsqrt(deg)
    enorm = dinv[src] * w * dinv[dst]
    snorm = dinv * dinv

    x_fm = jnp.zeros((128, NP), jnp.float32).at[:, :N_NODES].set(x.T)

    def prop_fm(H):
        out = snorm[None, :] * H
        return out.at[:, dst].add(enorm[None, :] * H[:, src])

    def hops3(H):
        for _ in range(3):
            H = prop_fm(H)
        return H

    T0 = _fm_matmul(W1, x_fm)
    H1 = hops3(T0)
    x1 = jnp.tanh(H1 + b1[:, None])
    T1 = _fm_matmul(W2, x1)
    H2 = hops3(T1)
    x2 = jnp.tanh(H2 + b2[:, None])
    T2 = _fm_matmul(W3, x2)
    H3 = hops3(T2)
    x3 = jnp.tanh(H3 + b3[:, None])

    h_fm = jnp.concatenate([x1, x2, x3], axis=0)

    key = x3[31, :N_NODES]
    batchp = batch
    counts = jnp.bincount(batchp, length=NUM_GRAPHS)
    starts = jnp.concatenate([jnp.zeros((1,), counts.dtype),
                              jnp.cumsum(counts)[:-1]])
    order = jnp.lexsort((-key, batchp))
    sb = batchp[order]
    rank = jnp.arange(N_NODES, dtype=counts.dtype) - starts[sb]
    validm = rank < K_POOL
    sel = jnp.full((K_POOL * NUM_GRAPHS,), -1, jnp.int32)
    slot = rank * NUM_GRAPHS + sb
    sel = sel.at[jnp.where(validm, slot, K_POOL * NUM_GRAPHS)].set(
        order.astype(jnp.int32), mode="drop")

    valid = sel >= 0
    pooled_fm = jnp.where(valid[None, :],
                          h_fm[:, jnp.where(valid, sel, 0)], 0.0)

    WcR = jnp.transpose(Wc, (1, 0, 2, 3)).reshape(K_POOL, 320, 96)
    Pm = pooled_fm.reshape(96, K_POOL, NUM_GRAPHS)
    priors = jnp.einsum('ikl,lib->ikb', WcR, Pm)

    pr_o = [priors[:, o * 32:(o + 1) * 32, :] for o in range(10)]
    logits = [jnp.zeros((K_POOL, NUM_GRAPHS), jnp.float32) for _ in range(10)]
    caps = None
    for t in range(NUM_ITER):
        m = logits[0]
        for o in range(1, 10):
            m = jnp.maximum(m, logits[o])
        es = [jnp.exp(l - m) for l in logits]
        ssum = es[0]
        for o in range(1, 10):
            ssum = ssum + es[o]
        probs = [e / ssum for e in es]
        caps = []
        for o in range(10):
            s = jnp.sum(probs[o][:, None, :] * pr_o[o], axis=0)
            sn = jnp.sum(s * s, axis=0, keepdims=True)
            caps.append((sn / (1.0 + sn)) * s / jnp.sqrt(sn + 1e-8))
        if t != NUM_ITER - 1:
            for o in range(10):
                dl = jnp.sum(pr_o[o] * caps[o][None, :, :], axis=1)
                logits[o] = logits[o] + dl
    classes = jnp.stack(
        [jnp.sqrt(jnp.sum(c * c, axis=0)) for c in caps], axis=1)
    return classes


# speedup vs baseline: 4.5198x; 4.4596x over previous
import functools

import jax
import jax.numpy as jnp
from jax import lax
from jax.experimental import pallas as pl
from jax.experimental.pallas import tpu as pltpu
from jax.experimental.pallas import tpu_sc as plsc

N_NODES = 10000
NP = 10240
N_EDGES = 320000
NUM_GRAPHS = 64
K_POOL = 50
NUM_ITER = 3
EC = 10000
N_TILES = 32

_mesh = plsc.VectorSubcoreMesh(core_axis_name="c", subcore_axis_name="s")


def _wid():
    return lax.axis_index("s") * 2 + lax.axis_index("c")


@functools.partial(
    pl.kernel, mesh=_mesh,
    compiler_params=pltpu.CompilerParams(needs_layout_passes=False),
    out_type=jax.ShapeDtypeStruct((2, NP), jnp.float32),
    scratch_types=[
        pltpu.VMEM((EC,), jnp.int32),
        pltpu.VMEM((EC,), jnp.int32),
        pltpu.VMEM((NP,), jnp.float32),
        pltpu.VMEM((16, 640), jnp.float32),
        pltpu.VMEM((640,), jnp.float32),
        pltpu.VMEM_SHARED((16, NP), jnp.float32),
    ],
)
def _sc_deg(src_hbm, dst_hbm, deg2, sbuf, dbuf, acc, red, rsl, spart):
    cid = lax.axis_index("c")
    sid = lax.axis_index("s")
    wid = sid * 2 + cid
    ones = jnp.ones((16,), jnp.float32)

    def zero_body(i, _):
        acc[pl.ds(i * 16, 16)] = jnp.zeros((16,), jnp.float32)
        return 0
    lax.fori_loop(0, NP // 16, zero_body, 0)

    pltpu.sync_copy(src_hbm.at[pl.ds(wid * EC, EC)], sbuf)
    pltpu.sync_copy(dst_hbm.at[pl.ds(wid * EC, EC)], dbuf)

    def edge_body(j, _):
        s16 = sbuf[pl.ds(j * 16, 16)]
        d16 = dbuf[pl.ds(j * 16, 16)]
        w = jnp.where(s16 != d16, ones, 0.0)
        plsc.addupdate_scatter(acc, [d16], w)
        return 0
    lax.fori_loop(0, EC // 16, edge_body, 0)

    pltpu.sync_copy(acc, spart.at[sid])
    plsc.subcore_barrier()
    pltpu.sync_copy(spart.at[:, pl.ds(sid * 640, 640)], red)

    def red_body(i, _):
        v = jnp.zeros((16,), jnp.float32)
        for r in range(16):
            v = v + red[r, pl.ds(i * 16, 16)]
        rsl[pl.ds(i * 16, 16)] = v
        return 0
    lax.fori_loop(0, 40, red_body, 0)
    pltpu.sync_copy(rsl, deg2.at[cid, pl.ds(sid * 640, 640)])


@functools.partial(
    pl.kernel, mesh=_mesh,
    compiler_params=pltpu.CompilerParams(needs_layout_passes=False),
    out_type=jax.ShapeDtypeStruct((N_TILES, 2, EC), jnp.int32),
    scratch_types=[
        pltpu.VMEM((EC,), jnp.int32),
        pltpu.VMEM((EC,), jnp.int32),
        pltpu.VMEM((NP,), jnp.float32),
        pltpu.VMEM((EC,), jnp.int32),
        pltpu.VMEM((EC,), jnp.int32),
    ],
)
def _sc_enorm(src_hbm, dst_hbm, dinv_hbm, edata, sbuf, dbuf, dv, pkb, enb):
    wid = _wid()
    pltpu.sync_copy(src_hbm.at[pl.ds(wid * EC, EC)], sbuf)
    pltpu.sync_copy(dst_hbm.at[pl.ds(wid * EC, EC)], dbuf)
    pltpu.sync_copy(dinv_hbm, dv)

    def body(j, _):
        s16 = sbuf[pl.ds(j * 16, 16)]
        d16 = dbuf[pl.ds(j * 16, 16)]
        ds_ = plsc.load_gather(dv, [s16])
        dd_ = plsc.load_gather(dv, [d16])
        en = jnp.where(s16 != d16, ds_ * dd_, 0.0)
        pkb[pl.ds(j * 16, 16)] = s16 | lax.shift_left(d16, 16)
        enb[pl.ds(j * 16, 16)] = plsc.bitcast(en, jnp.int32)
        return 0
    lax.fori_loop(0, EC // 16, body, 0)
    pltpu.sync_copy(pkb, edata.at[wid, 0])
    pltpu.sync_copy(enb, edata.at[wid, 1])


@functools.partial(
    pl.kernel, mesh=_mesh,
    compiler_params=pltpu.CompilerParams(needs_layout_passes=False),
    out_type=jax.ShapeDtypeStruct((N_TILES, NP), jnp.float32),
    scratch_types=[
        pltpu.VMEM((NP,), jnp.float32),
        pltpu.VMEM((NP,), jnp.float32),
        pltpu.VMEM((NP,), jnp.float32),
        pltpu.VMEM((2, EC), jnp.int32),
        pltpu.VMEM((2, EC), jnp.int32),
        pltpu.SemaphoreType.DMA,
        pltpu.SemaphoreType.DMA,
    ],
)
def _sc_prop3(h_hbm, edata_hbm, snorm_hbm, out_hbm,
              b0, b1, snv, ebA, ebB, semA, semB):
    wid = _wid()
    pltpu.sync_copy(h_hbm.at[wid], b0)
    pltpu.sync_copy(snorm_hbm, snv)

    def chunk_compute(eb, hs, hd):
        def body(j, _):
            pk = eb[0, pl.ds(j * 16, 16)]
            en = plsc.bitcast(eb[1, pl.ds(j * 16, 16)], jnp.float32)
            s16 = pk & 0xFFFF
            d16 = lax.shift_right_logical(pk, 16)
            vals = plsc.load_gather(hs, [s16])
            plsc.addupdate_scatter(hd, [d16], vals * en)
            return 0
        lax.fori_loop(0, EC // 16, body, 0)

    def hop(hs, hd):
        def init_body(i, _):
            sl = pl.ds(i * 16, 16)
            hd[sl] = snv[sl] * hs[sl]
            return 0
        lax.fori_loop(0, NP // 16, init_body, 0)

        cpA0 = pltpu.make_async_copy(edata_hbm.at[0], ebA, semA)
        cpA0.start()

        def outer(t, _):
            ca = 2 * t
            pltpu.make_async_copy(edata_hbm.at[ca], ebA, semA).wait()
            cpB = pltpu.make_async_copy(edata_hbm.at[ca + 1], ebB, semB)
            cpB.start()
            chunk_compute(ebA, hs, hd)
            pltpu.make_async_copy(edata_hbm.at[ca + 1], ebB, semB).wait()
            nxt = jnp.minimum(ca + 2, N_TILES - 2)
            pltpu.make_async_copy(edata_hbm.at[nxt], ebA, semA).start()
            chunk_compute(ebB, hs, hd)
            return 0
        lax.fori_loop(0, N_TILES // 2, outer, 0)
        pltpu.make_async_copy(edata_hbm.at[0], ebA, semA).wait()

    hop(b0, b1)
    hop(b1, b0)
    hop(b0, b1)
    pltpu.sync_copy(b1, out_hbm.at[wid])


@functools.partial(
    pl.kernel, mesh=_mesh,
    compiler_params=pltpu.CompilerParams(needs_layout_passes=False),
    out_type=jax.ShapeDtypeStruct((128, NP), jnp.float32),
    scratch_types=[
        pltpu.VMEM((4, NP), jnp.float32),
        pltpu.VMEM((4, NP), jnp.float32),
        pltpu.VMEM((NP,), jnp.float32),
        pltpu.VMEM((2, EC), jnp.int32),
    ],
)
def _sc_prop3x4(h_hbm, edata_hbm, snorm_hbm, out_hbm, b0, b1, snv, eb):
    wid = _wid()
    pltpu.sync_copy(h_hbm.at[pl.ds(4 * wid, 4)], b0)
    pltpu.sync_copy(snorm_hbm, snv)
    fful = [jnp.full((16,), ff, jnp.int32) for ff in range(4)]

    def hop(hs, hd):
        def init_body(i, _):
            sl = pl.ds(i * 16, 16)
            sv = snv[sl]
            for ff in range(4):
                hd[ff, sl] = sv * hs[ff, sl]
            return 0
        lax.fori_loop(0, NP // 16, init_body, 0)

        def chunk(c, _):
            pltpu.sync_copy(edata_hbm.at[c], eb)

            def body(j, _):
                pk = eb[0, pl.ds(j * 16, 16)]
                en = plsc.bitcast(eb[1, pl.ds(j * 16, 16)], jnp.float32)
                s16 = pk & 0xFFFF
                d16 = lax.shift_right_logical(pk, 16)
                for ff in range(4):
                    vals = plsc.load_gather(hs, [fful[ff], s16])
                    plsc.addupdate_scatter(hd, [fful[ff], d16], vals * en)
                return 0
            lax.fori_loop(0, EC // 16, body, 0)
            return 0
        lax.fori_loop(0, N_TILES, chunk, 0)

    hop(b0, b1)
    hop(b1, b0)
    hop(b0, b1)
    pltpu.sync_copy(b1, out_hbm.at[pl.ds(4 * wid, 4)])


def _mmbf_kernel(w_ref, x_ref, b_ref, o_ref):
    wb = w_ref[...].astype(jnp.bfloat16)
    xb = x_ref[...].astype(jnp.bfloat16)
    acc = jnp.dot(wb, xb, preferred_element_type=jnp.float32)
    o_ref[...] = jnp.tanh(acc + b_ref[...][:, 0:1])


def _fm_layer(W, X, b):
    fo, fi = W.shape
    blk = 1280
    b2d = jnp.broadcast_to(b[:, None], (fo, 128))
    return pl.pallas_call(
        _mmbf_kernel,
        grid=(NP // blk,),
        in_specs=[
            pl.BlockSpec((fo, fi), lambda j: (0, 0)),
            pl.BlockSpec((fi, blk), lambda j: (0, j)),
            pl.BlockSpec((fo, 128), lambda j: (0, 0)),
        ],
        out_specs=pl.BlockSpec((fo, blk), lambda j: (0, j)),
        out_shape=jax.ShapeDtypeStruct((fo, NP), jnp.float32),
    )(W, X, b2d)


def kernel(x, edge_index, batch, W1, b1, W2, b2, W3, b3, Wc):
    src, dst = edge_index[0], edge_index[1]

    deg2 = _sc_deg(src, dst)
    deg = deg2[0] + deg2[1] + 1.0
    dinv = jax.lax.rsqrt(deg)
    snorm = dinv * dinv

    edata = _sc_enorm(src, dst, dinv)

    x_fm = jnp.zeros((128, NP), jnp.float32).at[:, :N_NODES].set(x.T)

    H1 = _sc_prop3x4(x_fm, edata, snorm)
    x1 = _fm_layer(W1, H1, b1)
    H2 = _sc_prop3(x1, edata, snorm)
    x2 = _fm_layer(W2, H2, b2)
    H3 = _sc_prop3(x2, edata, snorm)
    x3 = _fm_layer(W3, H3, b3)

    h_fm = jnp.concatenate([x1, x2, x3], axis=0)

    key = x3[31, :N_NODES]
    counts = jnp.bincount(batch, length=NUM_GRAPHS)
    starts = jnp.concatenate([jnp.zeros((1,), counts.dtype),
                              jnp.cumsum(counts)[:-1]])
    order = jnp.lexsort((-key, batch))
    sb = batch[order]
    rank = jnp.arange(N_NODES, dtype=counts.dtype) - starts[sb]
    validm = rank < K_POOL
    sel = jnp.full((K_POOL * NUM_GRAPHS,), -1, jnp.int32)
    slot = rank * NUM_GRAPHS + sb
    sel = sel.at[jnp.where(validm, slot, K_POOL * NUM_GRAPHS)].set(
        order.astype(jnp.int32), mode="drop")

    valid = sel >= 0
    pooled_fm = jnp.where(valid[None, :],
                          h_fm[:, jnp.where(valid, sel, 0)], 0.0)

    WcR = jnp.transpose(Wc, (1, 0, 2, 3)).reshape(K_POOL, 320, 96)
    Pm = pooled_fm.reshape(96, K_POOL, NUM_GRAPHS)
    priors = jnp.einsum('ikl,lib->ikb', WcR.astype(jnp.bfloat16),
                        Pm.astype(jnp.bfloat16),
                        preferred_element_type=jnp.float32)

    pr_o = [priors[:, o * 32:(o + 1) * 32, :] for o in range(10)]
    logits = [jnp.zeros((K_POOL, NUM_GRAPHS), jnp.float32) for _ in range(10)]
    caps = None
    for t in range(NUM_ITER):
        m = logits[0]
        for o in range(1, 10):
            m = jnp.maximum(m, logits[o])
        es = [jnp.exp(l - m) for l in logits]
        ssum = es[0]
        for o in range(1, 10):
            ssum = ssum + es[o]
        probs = [e / ssum for e in es]
        caps = []
        for o in range(10):
            s = jnp.sum(probs[o][:, None, :] * pr_o[o], axis=0)
            sn = jnp.sum(s * s, axis=0, keepdims=True)
            caps.append((sn / (1.0 + sn)) * s / jnp.sqrt(sn + 1e-8))
        if t != NUM_ITER - 1:
            for o in range(10):
                dl = jnp.sum(pr_o[o] * caps[o][None, :, :], axis=1)
                logits[o] = logits[o] + dl
    classes = jnp.stack(
        [jnp.sqrt(jnp.sum(c * c, axis=0)) for c in caps], axis=1)
    return classes


# speedup vs baseline: 13.0619x; 2.8899x over previous
import functools

import jax
import jax.numpy as jnp
from jax import lax
from jax.experimental import pallas as pl
from jax.experimental.pallas import tpu as pltpu
from jax.experimental.pallas import tpu_sc as plsc

N_NODES = 10000
NP = 10240
N_EDGES = 320000
NUM_GRAPHS = 64
K_POOL = 50
NUM_ITER = 3
EC = 10000
N_TILES = 32

_mesh = plsc.VectorSubcoreMesh(core_axis_name="c", subcore_axis_name="s")


def _wid():
    return lax.axis_index("s") * 2 + lax.axis_index("c")


@functools.partial(
    pl.kernel, mesh=_mesh,
    compiler_params=pltpu.CompilerParams(needs_layout_passes=False),
    out_type=jax.ShapeDtypeStruct((2, NP), jnp.float32),
    scratch_types=[
        pltpu.VMEM((EC,), jnp.int32),
        pltpu.VMEM((EC,), jnp.int32),
        pltpu.VMEM((NP,), jnp.float32),
        pltpu.VMEM((16, 640), jnp.float32),
        pltpu.VMEM((640,), jnp.float32),
        pltpu.VMEM_SHARED((16, NP), jnp.float32),
    ],
)
def _sc_deg(src_hbm, dst_hbm, deg2, sbuf, dbuf, acc, red, rsl, spart):
    cid = lax.axis_index("c")
    sid = lax.axis_index("s")
    wid = sid * 2 + cid
    ones = jnp.ones((16,), jnp.float32)

    def zero_body(i, _):
        acc[pl.ds(i * 16, 16)] = jnp.zeros((16,), jnp.float32)
        return 0
    lax.fori_loop(0, NP // 16, zero_body, 0)

    pltpu.sync_copy(src_hbm.at[pl.ds(wid * EC, EC)], sbuf)
    pltpu.sync_copy(dst_hbm.at[pl.ds(wid * EC, EC)], dbuf)

    def edge_body(j, _):
        s16 = sbuf[pl.ds(j * 16, 16)]
        d16 = dbuf[pl.ds(j * 16, 16)]
        w = jnp.where(s16 != d16, ones, 0.0)
        plsc.addupdate_scatter(acc, [d16], w)
        return 0
    lax.fori_loop(0, EC // 16, edge_body, 0)

    pltpu.sync_copy(acc, spart.at[sid])
    plsc.subcore_barrier()
    pltpu.sync_copy(spart.at[:, pl.ds(sid * 640, 640)], red)

    def red_body(i, _):
        v = jnp.zeros((16,), jnp.float32)
        for r in range(16):
            v = v + red[r, pl.ds(i * 16, 16)]
        rsl[pl.ds(i * 16, 16)] = v
        return 0
    lax.fori_loop(0, 40, red_body, 0)
    pltpu.sync_copy(rsl, deg2.at[cid, pl.ds(sid * 640, 640)])


@functools.partial(
    pl.kernel, mesh=_mesh,
    compiler_params=pltpu.CompilerParams(needs_layout_passes=False),
    out_type=jax.ShapeDtypeStruct((N_TILES, 2, EC), jnp.int32),
    scratch_types=[
        pltpu.VMEM((EC,), jnp.int32),
        pltpu.VMEM((EC,), jnp.int32),
        pltpu.VMEM((NP,), jnp.float32),
        pltpu.VMEM((EC,), jnp.int32),
        pltpu.VMEM((EC,), jnp.int32),
    ],
)
def _sc_enorm(src_hbm, dst_hbm, dinv_hbm, edata, sbuf, dbuf, dv, pkb, enb):
    wid = _wid()
    pltpu.sync_copy(src_hbm.at[pl.ds(wid * EC, EC)], sbuf)
    pltpu.sync_copy(dst_hbm.at[pl.ds(wid * EC, EC)], dbuf)
    pltpu.sync_copy(dinv_hbm, dv)

    def body(j, _):
        s16 = sbuf[pl.ds(j * 16, 16)]
        d16 = dbuf[pl.ds(j * 16, 16)]
        ds_ = plsc.load_gather(dv, [s16])
        dd_ = plsc.load_gather(dv, [d16])
        en = jnp.where(s16 != d16, ds_ * dd_, 0.0)
        pkb[pl.ds(j * 16, 16)] = s16 | lax.shift_left(d16, 16)
        enb[pl.ds(j * 16, 16)] = plsc.bitcast(en, jnp.int32)
        return 0
    lax.fori_loop(0, EC // 16, body, 0)
    pltpu.sync_copy(pkb, edata.at[wid, 0])
    pltpu.sync_copy(enb, edata.at[wid, 1])


@functools.partial(
    pl.kernel, mesh=_mesh,
    compiler_params=pltpu.CompilerParams(needs_layout_passes=False),
    out_type=jax.ShapeDtypeStruct((N_TILES, NP), jnp.float32),
    scratch_types=[
        pltpu.VMEM((NP,), jnp.float32),
        pltpu.VMEM((NP,), jnp.float32),
        pltpu.VMEM((NP,), jnp.float32),
        pltpu.VMEM((2, EC), jnp.int32),
        pltpu.VMEM((2, EC), jnp.int32),
        pltpu.SemaphoreType.DMA,
        pltpu.SemaphoreType.DMA,
    ],
)
def _sc_prop3(h_hbm, edata_hbm, snorm_hbm, out_hbm,
              b0, b1, snv, ebA, ebB, semA, semB):
    wid = _wid()
    pltpu.sync_copy(h_hbm.at[wid], b0)
    pltpu.sync_copy(snorm_hbm, snv)

    def chunk_compute(eb, hs, hd):
        @plsc.parallel_loop(0, EC // 16, unroll=8)
        def body(j):
            pk = eb[0, pl.ds(j * 16, 16)]
            en = plsc.bitcast(eb[1, pl.ds(j * 16, 16)], jnp.float32)
            s16 = pk & 0xFFFF
            d16 = lax.shift_right_logical(pk, 16)
            vals = plsc.load_gather(hs, [s16])
            plsc.addupdate_scatter(hd, [d16], vals * en)

    def hop(hs, hd):
        @plsc.parallel_loop(0, NP // 16, unroll=4)
        def init_body(i):
            sl = pl.ds(i * 16, 16)
            hd[sl] = snv[sl] * hs[sl]

        cpA0 = pltpu.make_async_copy(edata_hbm.at[0], ebA, semA)
        cpA0.start()

        def outer(t, _):
            ca = 2 * t
            pltpu.make_async_copy(edata_hbm.at[ca], ebA, semA).wait()
            cpB = pltpu.make_async_copy(edata_hbm.at[ca + 1], ebB, semB)
            cpB.start()
            chunk_compute(ebA, hs, hd)
            pltpu.make_async_copy(edata_hbm.at[ca + 1], ebB, semB).wait()
            nxt = jnp.minimum(ca + 2, N_TILES - 2)
            pltpu.make_async_copy(edata_hbm.at[nxt], ebA, semA).start()
            chunk_compute(ebB, hs, hd)
            return 0
        lax.fori_loop(0, N_TILES // 2, outer, 0)
        pltpu.make_async_copy(edata_hbm.at[0], ebA, semA).wait()

    hop(b0, b1)
    hop(b1, b0)
    hop(b0, b1)
    pltpu.sync_copy(b1, out_hbm.at[wid])


@functools.partial(
    pl.kernel, mesh=_mesh,
    compiler_params=pltpu.CompilerParams(needs_layout_passes=False),
    out_type=jax.ShapeDtypeStruct((128, NP), jnp.float32),
    scratch_types=[
        pltpu.VMEM((4, NP), jnp.float32),
        pltpu.VMEM((4, NP), jnp.float32),
        pltpu.VMEM((NP,), jnp.float32),
        pltpu.VMEM((2, EC), jnp.int32),
    ],
)
def _sc_prop3x4(h_hbm, edata_hbm, snorm_hbm, out_hbm, b0, b1, snv, eb):
    wid = _wid()
    pltpu.sync_copy(h_hbm.at[pl.ds(4 * wid, 4)], b0)
    pltpu.sync_copy(snorm_hbm, snv)
    fful = [jnp.full((16,), ff, jnp.int32) for ff in range(4)]

    def hop(hs, hd):
        @plsc.parallel_loop(0, NP // 16, unroll=4)
        def init_body(i):
            sl = pl.ds(i * 16, 16)
            sv = snv[sl]
            for ff in range(4):
                hd[ff, sl] = sv * hs[ff, sl]

        def chunk(c, _):
            pltpu.sync_copy(edata_hbm.at[c], eb)

            @plsc.parallel_loop(0, EC // 16, unroll=4)
            def body(j):
                pk = eb[0, pl.ds(j * 16, 16)]
                en = plsc.bitcast(eb[1, pl.ds(j * 16, 16)], jnp.float32)
                s16 = pk & 0xFFFF
                d16 = lax.shift_right_logical(pk, 16)
                for ff in range(4):
                    vals = plsc.load_gather(hs, [fful[ff], s16])
                    plsc.addupdate_scatter(hd, [fful[ff], d16], vals * en)
            return 0
        lax.fori_loop(0, N_TILES, chunk, 0)

    hop(b0, b1)
    hop(b1, b0)
    hop(b0, b1)
    pltpu.sync_copy(b1, out_hbm.at[pl.ds(4 * wid, 4)])


def _mmbf_kernel(w_ref, x_ref, b_ref, o_ref):
    wb = w_ref[...].astype(jnp.bfloat16)
    xb = x_ref[...].astype(jnp.bfloat16)
    acc = jnp.dot(wb, xb, preferred_element_type=jnp.float32)
    o_ref[...] = jnp.tanh(acc + b_ref[...][:, 0:1])


def _fm_layer(W, X, b):
    fo, fi = W.shape
    blk = 1280
    b2d = jnp.broadcast_to(b[:, None], (fo, 128))
    return pl.pallas_call(
        _mmbf_kernel,
        grid=(NP // blk,),
        in_specs=[
            pl.BlockSpec((fo, fi), lambda j: (0, 0)),
            pl.BlockSpec((fi, blk), lambda j: (0, j)),
            pl.BlockSpec((fo, 128), lambda j: (0, 0)),
        ],
        out_specs=pl.BlockSpec((fo, blk), lambda j: (0, j)),
        out_shape=jax.ShapeDtypeStruct((fo, NP), jnp.float32),
    )(W, X, b2d)


def kernel(x, edge_index, batch, W1, b1, W2, b2, W3, b3, Wc):
    src, dst = edge_index[0], edge_index[1]

    deg2 = _sc_deg(src, dst)
    deg = deg2[0] + deg2[1] + 1.0
    dinv = jax.lax.rsqrt(deg)
    snorm = dinv * dinv

    edata = _sc_enorm(src, dst, dinv)

    x_fm = jnp.zeros((128, NP), jnp.float32).at[:, :N_NODES].set(x.T)

    H1 = _sc_prop3x4(x_fm, edata, snorm)
    x1 = _fm_layer(W1, H1, b1)
    H2 = _sc_prop3(x1, edata, snorm)
    x2 = _fm_layer(W2, H2, b2)
    H3 = _sc_prop3(x2, edata, snorm)
    x3 = _fm_layer(W3, H3, b3)

    h_fm = jnp.concatenate([x1, x2, x3], axis=0)

    key = x3[31, :N_NODES]
    counts = jnp.bincount(batch, length=NUM_GRAPHS)
    starts = jnp.concatenate([jnp.zeros((1,), counts.dtype),
                              jnp.cumsum(counts)[:-1]])
    order = jnp.lexsort((-key, batch))
    sb = batch[order]
    rank = jnp.arange(N_NODES, dtype=counts.dtype) - starts[sb]
    validm = rank < K_POOL
    sel = jnp.full((K_POOL * NUM_GRAPHS,), -1, jnp.int32)
    slot = rank * NUM_GRAPHS + sb
    sel = sel.at[jnp.where(validm, slot, K_POOL * NUM_GRAPHS)].set(
        order.astype(jnp.int32), mode="drop")

    valid = sel >= 0
    pooled_fm = jnp.where(valid[None, :],
                          h_fm[:, jnp.where(valid, sel, 0)], 0.0)

    WcR = jnp.transpose(Wc, (1, 0, 2, 3)).reshape(K_POOL, 320, 96)
    Pm = pooled_fm.reshape(96, K_POOL, NUM_GRAPHS)
    priors = jnp.einsum('ikl,lib->ikb', WcR.astype(jnp.bfloat16),
                        Pm.astype(jnp.bfloat16),
                        preferred_element_type=jnp.float32)

    pr_o = [priors[:, o * 32:(o + 1) * 32, :] for o in range(10)]
    logits = [jnp.zeros((K_POOL, NUM_GRAPHS), jnp.float32) for _ in range(10)]
    caps = None
    for t in range(NUM_ITER):
        m = logits[0]
        for o in range(1, 10):
            m = jnp.maximum(m, logits[o])
        es = [jnp.exp(l - m) for l in logits]
        ssum = es[0]
        for o in range(1, 10):
            ssum = ssum + es[o]
        probs = [e / ssum for e in es]
        caps = []
        for o in range(10):
            s = jnp.sum(probs[o][:, None, :] * pr_o[o], axis=0)
            sn = jnp.sum(s * s, axis=0, keepdims=True)
            caps.append((sn / (1.0 + sn)) * s / jnp.sqrt(sn + 1e-8))
        if t != NUM_ITER - 1:
            for o in range(10):
                dl = jnp.sum(pr_o[o] * caps[o][None, :, :], axis=1)
                logits[o] = logits[o] + dl
    classes = jnp.stack(
        [jnp.sqrt(jnp.sum(c * c, axis=0)) for c in caps], axis=1)
    return classes
